# baseline (speedup 1.0000x reference)
import functools
import math

import jax
import jax.numpy as jnp
from jax import lax
from jax.experimental import pallas as pl
from jax.experimental.pallas import tpu as pltpu
from jax.experimental.pallas import tpu_sc as plsc


def kernel(x, table):
    B0, S = x.shape
    V, D = table.shape
    W = 2 * D
    scale = math.sqrt(D)

    info = plsc.get_sparse_core_info()
    NC, NS, L = info.num_cores, info.num_subcores, info.num_lanes
    NW = NC * NS

    VB = 128
    n_full = V // VB
    tail = V - n_full * VB
    per_w = n_full // NW
    n_extra = n_full - per_w * NW

    mesh = plsc.VectorSubcoreMesh(core_axis_name="c", subcore_axis_name="s")
    cparams = pltpu.CompilerParams(
        use_tc_tiling_on_sc=True, needs_layout_passes=False
    )

    @functools.partial(
        pl.kernel,
        mesh=mesh,
        out_type=jax.ShapeDtypeStruct((V // 2, W), jnp.float32),
        scratch_types=[
            pltpu.VMEM((D, VB), jnp.float32),
            pltpu.VMEM((VB // 2, W), jnp.float32),
            pltpu.VMEM((D, D), jnp.float32),
        ],
        compiler_params=cparams,
    )
    def pack(tt_hbm, out_hbm, src_v, tr_v, tail_v):
        wid = lax.axis_index("s") * NC + lax.axis_index("c")
        lane = lax.iota(jnp.int32, L)

        def do_block(blk):
            v0 = blk * VB
            pltpu.sync_copy(tt_hbm.at[:, pl.ds(v0, VB)], src_v)

            def v_body(vl, carry):
                row = jnp.full((L,), 0, jnp.int32)
                for dc in range(D // L):
                    vals = plsc.load_gather(
                        src_v, [dc * L + lane, jnp.full((L,), vl, jnp.int32)]
                    )
                    u = vl // 2
                    half = (vl % 2) * D
                    tr_v[u, pl.ds(half + dc * L, L)] = vals * scale
                return carry

            lax.fori_loop(0, VB, v_body, 0, unroll=2)
            pltpu.sync_copy(tr_v, out_hbm.at[pl.ds(blk * (VB // 2), VB // 2)])

        def blk_body(c, carry):
            do_block(wid * per_w + c)
            return carry

        lax.fori_loop(0, per_w, blk_body, 0)

        @pl.when(wid < n_extra)
        def _():
            do_block(NW * per_w + wid)

        @pl.when(wid == n_extra)
        def _():
            v0 = n_full * VB

            def d_body(d, carry):
                pltpu.sync_copy(tt_hbm.at[d, pl.ds(v0, tail)], tail_v.at[d])
                return carry

            lax.fori_loop(0, D, d_body, 0)

            def v_body(vl, carry):
                for dc in range(D // L):
                    vals = plsc.load_gather(
                        tail_v, [dc * L + lane, jnp.full((L,), vl, jnp.int32)]
                    )
                    u = vl // 2
                    half = (vl % 2) * D
                    tr_v[u, pl.ds(half + dc * L, L)] = vals * scale
                return carry

            lax.fori_loop(0, tail, v_body, 0, unroll=2)
            pltpu.sync_copy(
                tr_v.at[pl.ds(0, tail // 2)],
                out_hbm.at[pl.ds(v0 // 2, tail // 2)],
            )

    SB = 4
    n_sb = S // SB
    BW = B0 // NW
    RG = SB * BW

    @functools.partial(
        pl.kernel,
        mesh=mesh,
        out_type=jax.ShapeDtypeStruct((S, D, B0), jnp.float32),
        scratch_types=[
            pltpu.VMEM((SB, BW), jnp.int32),
            pltpu.VMEM((SB, BW), jnp.int32),
            pltpu.VMEM((RG, W), jnp.float32),
            pltpu.VMEM((SB, D, BW), jnp.float32),
            pltpu.SemaphoreType.DMA,
        ],
        compiler_params=cparams,
    )
    def emb(tp_hbm, idxh_hbm, par_hbm, out_hbm, idx_v, par_v, g_v, t_v, sem):
        wid = lax.axis_index("s") * NC + lax.axis_index("c")
        b0 = wid * BW
        lane = lax.iota(jnp.int32, L)

        def sb_body(sb, carry):
            pltpu.sync_copy(idxh_hbm.at[sb, :, pl.ds(b0, BW)], idx_v)
            pltpu.sync_copy(par_hbm.at[sb, :, pl.ds(b0, BW)], par_v)
            for sl in range(SB):
                pltpu.async_copy(
                    tp_hbm.at[idx_v.at[sl]], g_v.at[pl.ds(sl * BW, BW)], sem
                ).wait()
            for sl in range(SB):
                for bc in range(BW // L):
                    row = sl * BW + bc * L + lane
                    colbase = par_v[sl, pl.ds(bc * L, L)] * D

                    def d_body(d, carry2):
                        vals = plsc.load_gather(g_v, [row, colbase + d])
                        t_v[sl, d, pl.ds(bc * L, L)] = vals
                        return carry2

                    lax.fori_loop(0, D, d_body, 0, unroll=4)
            pltpu.sync_copy(
                t_v, out_hbm.at[pl.ds(sb * SB, SB), :, pl.ds(b0, BW)]
            )
            return carry

        lax.fori_loop(0, n_sb, sb_body, 0)

    tpairs = pack(table.T)
    xT = x.T
    idxh3 = (xT >> 1).reshape(n_sb, SB, B0)
    par3 = (xT & 1).reshape(n_sb, SB, B0)
    out3 = emb(tpairs, idxh3, par3)
    return out3.transpose(2, 0, 1)
